# baseline (speedup 1.0000x reference)
import functools

import jax
import jax.numpy as jnp
from jax import lax
from jax.experimental import pallas as pl
from jax.experimental.pallas import tpu as pltpu
from jax.experimental.pallas import tpu_sc as plsc

N = 10000
NFEAT = 512
NHID = 128
NOUT = 64
K = 100
KP = 112
NP = 10016
NC = 10240

_ROW_BLK = 2000
_SIM_BLK = 512

_NW = 32
_G = 40
_LS = 49
_L1PAD = 800


def _mlp_kernel(f_ref, w1_ref, b1_ref, w2_ref, b2_ref, emb_ref, xn_ref):
    h = jnp.maximum(
        jnp.dot(f_ref[...], w1_ref[...], preferred_element_type=jnp.float32)
        + b1_ref[...], 0.0)
    emb = jnp.dot(h, w2_ref[...], preferred_element_type=jnp.float32) + b2_ref[...]
    emb_ref[...] = emb
    nrm = jnp.sqrt(jnp.sum(emb * emb, axis=1, keepdims=True))
    xn_ref[...] = emb / jnp.clip(nrm, 1e-12, None)


def _sim_kernel(xb_ref, xf_ref, sim_ref):
    s = lax.dot_general(
        xb_ref[...], xf_ref[...],
        dimension_numbers=(((1,), (1,)), ((), ())),
        preferred_element_type=jnp.float32)
    col = lax.broadcasted_iota(jnp.int32, s.shape, 1)
    sim_ref[...] = jnp.where(col < N, s, -2.0)


def _norm_kernel(p_ref, rn_ref):
    rn_ref[...] = lax.rsqrt(jnp.sum(p_ref[...], axis=0, keepdims=True))


def _mlp(features, W1, b1, W2, b2):
    return pl.pallas_call(
        _mlp_kernel,
        grid=(N // _ROW_BLK,),
        in_specs=[
            pl.BlockSpec((_ROW_BLK, NFEAT), lambda i: (i, 0)),
            pl.BlockSpec((NFEAT, NHID), lambda i: (0, 0)),
            pl.BlockSpec((NHID,), lambda i: (0,)),
            pl.BlockSpec((NHID, NOUT), lambda i: (0, 0)),
            pl.BlockSpec((NOUT,), lambda i: (0,)),
        ],
        out_specs=[
            pl.BlockSpec((_ROW_BLK, NOUT), lambda i: (i, 0)),
            pl.BlockSpec((_ROW_BLK, NOUT), lambda i: (i, 0)),
        ],
        out_shape=[
            jax.ShapeDtypeStruct((N, NOUT), jnp.float32),
            jax.ShapeDtypeStruct((N, NOUT), jnp.float32),
        ],
    )(features, W1, b1, W2, b2)


def _sim(xn_pad):
    return pl.pallas_call(
        _sim_kernel,
        grid=(NC // _SIM_BLK,),
        in_specs=[
            pl.BlockSpec((_SIM_BLK, NOUT), lambda i: (i, 0)),
            pl.BlockSpec((NC, NOUT), lambda i: (0, 0)),
        ],
        out_specs=pl.BlockSpec((_SIM_BLK, NC), lambda i: (i, 0)),
        out_shape=jax.ShapeDtypeStruct((NC, NC), jnp.float32),
    )(xn_pad, xn_pad)


def _norm(partials):
    return pl.pallas_call(
        _norm_kernel,
        out_shape=jax.ShapeDtypeStruct((1, NP), jnp.float32),
    )(partials)


def _iota16():
    return lax.iota(jnp.int32, 16)


def _splat(x, dtype):
    return jnp.full((16,), x, dtype=dtype)


def _lane_of(mask):
    return jnp.max(plsc.all_reduce_ffs(mask))


def _extract(vec, l, lowest):
    return jnp.max(jnp.where(_iota16() == l, vec, lowest))


def _pick(row_v, l1t_v, vals_v, inds_v, t, colmax, colarg):
    iota = _iota16()
    lane0 = iota == 0
    m = jnp.max(colmax)
    l = _lane_of(colmax == m)
    g = _extract(colarg, l, 0)
    c = g * 16 + l
    v = row_v[pl.ds(c * 16, 16)]
    k = _lane_of(v == m)
    src = c * 16 + k
    plsc.store_scatter(vals_v, [_splat(t, jnp.int32)],
                       _splat(m, jnp.float32), mask=lane0)
    plsc.store_scatter(inds_v, [_splat(t, jnp.int32)],
                       _splat(src, jnp.int32), mask=lane0)
    vnew = jnp.where(iota == k, -3.0, v)
    row_v[pl.ds(c * 16, 16)] = vnew
    plsc.store_scatter(l1t_v, [_splat(l * _LS + g, jnp.int32)],
                       _splat(jnp.max(vnew), jnp.float32), mask=lane0)
    c0 = l1t_v[pl.ds(l * _LS, 16)]
    c1 = l1t_v[pl.ds(l * _LS + 16, 16)]
    c2 = l1t_v[pl.ds(l * _LS + 32, 16)]
    best = jnp.max(jnp.maximum(jnp.maximum(c0, c1), c2))
    gb = jnp.min(jnp.where(c0 == best, iota,
                 jnp.where(c1 == best, iota + 16,
                 jnp.where(c2 == best, iota + 32, 99))))
    at_l = iota == l
    return jnp.where(at_l, best, colmax), jnp.where(at_l, gb, colarg)


def _accum_degrees(vals_v, inds_v, part_v, r, live):
    iota = _iota16()
    rowsum = jnp.zeros((), jnp.float32)
    for c in range(KP // 16):
        vv = vals_v[pl.ds(c * 16, 16)]
        ii = inds_v[pl.ds(c * 16, 16)]
        valid = jnp.logical_and((c * 16 + iota) < (K + 1), live)
        plsc.addupdate_scatter(part_v, [ii], vv, mask=valid)
        rowsum = rowsum + jnp.sum(jnp.where(valid, vv, 0.0))
    plsc.addupdate_scatter(part_v, [_splat(r, jnp.int32)],
                           _splat(rowsum, jnp.float32),
                           mask=jnp.logical_and(iota == 0, live))


_RIF = 6


def _sc_topk_body(sim_hbm, vals_hbm, inds_hbm, part_hbm, *scr):
    rows = scr[0:_RIF]
    l1ts = scr[_RIF:2 * _RIF]
    valss = scr[2 * _RIF:3 * _RIF]
    indss = scr[3 * _RIF:4 * _RIF]
    part_v = scr[4 * _RIF]

    wid = lax.axis_index("s") * 2 + lax.axis_index("c")
    zf = jnp.zeros((16,), jnp.float32)
    zi = jnp.zeros((16,), jnp.int32)

    for s in range(_RIF):
        for c in range(KP // 16):
            valss[s][pl.ds(c * 16, 16)] = zf
            indss[s][pl.ds(c * 16, 16)] = zi
        for c in range(0, _L1PAD, 16):
            l1ts[s][pl.ds(c, 16)] = jnp.full((16,), -4.0, jnp.float32)

    def _zero(i, _):
        part_v[pl.ds(i * 16, 16)] = zf
        return 0
    lax.fori_loop(0, NP // 16, _zero, 0)

    def _do_batch(i, wid):
        r = [wid + _NW * (_RIF * i + s) for s in range(_RIF)]
        for s in range(_RIF):
            pltpu.sync_copy(sim_hbm.at[r[s]], rows[s])

        iota = _iota16()

        def _build(g, carry):
            out = []
            for s in range(_RIF):
                cmax, carg = carry[2 * s], carry[2 * s + 1]
                acc = jnp.full((16,), -4.0, jnp.float32)
                for l in range(16):
                    v = rows[s][pl.ds(g * 256 + l * 16, 16)]
                    acc = jnp.where(iota == l, jnp.max(v), acc)
                plsc.store_scatter(l1ts[s], [iota * _LS + _splat(g, jnp.int32)],
                                   acc)
                upd = acc > cmax
                out.extend((jnp.maximum(cmax, acc), jnp.where(upd, g, carg)))
            return tuple(out)
        st = lax.fori_loop(
            0, _G, _build,
            tuple([jnp.full((16,), -5.0, jnp.float32),
                   jnp.zeros((16,), jnp.int32)] * _RIF))

        def _select(t, carry):
            out = []
            for s in range(_RIF):
                cm, ca = _pick(rows[s], l1ts[s], valss[s], indss[s], t,
                               carry[2 * s], carry[2 * s + 1])
                out.extend((cm, ca))
            return tuple(out)
        lax.fori_loop(0, K + 1, _select, tuple(st))

        for s in range(_RIF):
            live = r[s] < N if s > 0 else True
            _accum_degrees(valss[s], indss[s], part_v, r[s], live)
            if s == 0:
                pltpu.sync_copy(valss[0], vals_hbm.at[r[0]])
                pltpu.sync_copy(indss[0], inds_hbm.at[r[0]])
            else:
                @pl.when(live)
                def _(s=s):
                    pltpu.sync_copy(valss[s], vals_hbm.at[r[s]])
                    pltpu.sync_copy(indss[s], inds_hbm.at[r[s]])
        return wid

    batches = jnp.where(wid < (N - (N // _NW) * _NW),
                        (N // _NW + 1 + _RIF - 1) // _RIF,
                        N // _NW // _RIF)
    lax.fori_loop(0, batches, _do_batch, wid)
    pltpu.sync_copy(part_v, part_hbm.at[wid])


def _sc_topk(sim):
    mesh = plsc.VectorSubcoreMesh(core_axis_name="c", subcore_axis_name="s")
    return pl.kernel(
        _sc_topk_body,
        out_type=[
            jax.ShapeDtypeStruct((N, KP), jnp.float32),
            jax.ShapeDtypeStruct((N, KP), jnp.int32),
            jax.ShapeDtypeStruct((_NW, NP), jnp.float32),
        ],
        mesh=mesh,
        compiler_params=pltpu.CompilerParams(needs_layout_passes=False),
        scratch_types=(
            [pltpu.VMEM((NC,), jnp.float32)] * _RIF
            + [pltpu.VMEM((_L1PAD,), jnp.float32)] * _RIF
            + [pltpu.VMEM((KP,), jnp.float32)] * _RIF
            + [pltpu.VMEM((KP,), jnp.int32)] * _RIF
            + [pltpu.VMEM((NP,), jnp.float32)]
        ),
    )(sim)


_EB = 8
_EBAND = 312


def _sc_edgeval_body(vals_hbm, inds_hbm, rn_hbm, out_hbm,
                     rn_v, vv_v, iv_v, ov_v, vv1_v, iv1_v, ov1_v):
    wid = lax.axis_index("s") * 2 + lax.axis_index("c")
    pltpu.sync_copy(rn_hbm.at[0], rn_v)
    lo = wid * _EBAND

    def _scale_row(vv, iv, ov, i, r):
        rr = plsc.load_gather(rn_v, [_splat(r, jnp.int32)])
        for c in range(KP // 16):
            if i is None:
                v = vv[pl.ds(c * 16, 16)]
                ii = iv[pl.ds(c * 16, 16)]
            else:
                v = vv[i, pl.ds(c * 16, 16)]
                ii = iv[i, pl.ds(c * 16, 16)]
            rc = plsc.load_gather(rn_v, [ii])
            if i is None:
                ov[pl.ds(c * 16, 16)] = v * rr * rc
            else:
                ov[i, pl.ds(c * 16, 16)] = v * rr * rc

    def _do_block(b, lo):
        base = lo + _EB * b
        pltpu.sync_copy(vals_hbm.at[pl.ds(base, _EB)], vv_v)
        pltpu.sync_copy(inds_hbm.at[pl.ds(base, _EB)], iv_v)
        for i in range(_EB):
            _scale_row(vv_v, iv_v, ov_v, i, base + i)
        pltpu.sync_copy(ov_v, out_hbm.at[pl.ds(base, _EB)])
        return lo

    lax.fori_loop(0, _EBAND // _EB, _do_block, lo)

    @pl.when(wid < N - _NW * _EBAND)
    def _():
        r = _NW * _EBAND + wid
        pltpu.sync_copy(vals_hbm.at[r], vv1_v)
        pltpu.sync_copy(inds_hbm.at[r], iv1_v)
        _scale_row(vv1_v, iv1_v, ov1_v, None, r)
        pltpu.sync_copy(ov1_v, out_hbm.at[r])


def _sc_edgeval(vals, inds, rnorm):
    mesh = plsc.VectorSubcoreMesh(core_axis_name="c", subcore_axis_name="s")
    return pl.kernel(
        _sc_edgeval_body,
        out_type=jax.ShapeDtypeStruct((N, KP), jnp.float32),
        mesh=mesh,
        compiler_params=pltpu.CompilerParams(needs_layout_passes=False),
        scratch_types=[
            pltpu.VMEM((NP,), jnp.float32),
            pltpu.VMEM((_EB, KP), jnp.float32),
            pltpu.VMEM((_EB, KP), jnp.int32),
            pltpu.VMEM((_EB, KP), jnp.float32),
            pltpu.VMEM((KP,), jnp.float32),
            pltpu.VMEM((KP,), jnp.int32),
            pltpu.VMEM((KP,), jnp.float32),
        ],
    )(vals, inds, rnorm)


def kernel(features, W1, b1, W2, b2):
    emb, xn = _mlp(features, W1, b1, W2, b2)
    xn_pad = jnp.pad(xn, ((0, NC - N), (0, 0)))
    sim = _sim(xn_pad)
    vals, inds, partials = _sc_topk(sim)
    rnorm = _norm(partials)
    values = _sc_edgeval(vals, inds, rnorm)

    rows = jnp.repeat(jnp.arange(N, dtype=jnp.int32), K + 1)
    cols = inds[:, :K + 1].reshape(-1)
    values = values[:, :K + 1].reshape(-1)
    edge_index = jnp.stack([rows.astype(jnp.int64), cols.astype(jnp.int64)],
                           axis=0)
    return edge_index, values, emb


# speedup vs baseline: 1.1179x; 1.1179x over previous
import functools

import jax
import jax.numpy as jnp
from jax import lax
from jax.experimental import pallas as pl
from jax.experimental.pallas import tpu as pltpu
from jax.experimental.pallas import tpu_sc as plsc

N = 10000
NFEAT = 512
NHID = 128
NOUT = 64
K = 100
KP = 112
NP = 10016
NC = 10240

_ROW_BLK = 2000
_SIM_BLK = 512

_NW = 32
_G = 40
_LS = 49
_L1PAD = 800


def _mlp_kernel(f_ref, w1_ref, b1_ref, w2_ref, b2_ref, emb_ref, xn_ref):
    h = jnp.maximum(
        jnp.dot(f_ref[...], w1_ref[...], preferred_element_type=jnp.float32)
        + b1_ref[...], 0.0)
    emb = jnp.dot(h, w2_ref[...], preferred_element_type=jnp.float32) + b2_ref[...]
    emb_ref[...] = emb
    nrm = jnp.sqrt(jnp.sum(emb * emb, axis=1, keepdims=True))
    xn_ref[...] = emb / jnp.clip(nrm, 1e-12, None)


def _sim_kernel(xb_ref, xf_ref, sim_ref):
    s = lax.dot_general(
        xb_ref[...], xf_ref[...],
        dimension_numbers=(((1,), (1,)), ((), ())),
        preferred_element_type=jnp.float32)
    col = lax.broadcasted_iota(jnp.int32, s.shape, 1)
    sim_ref[...] = jnp.where(col < N, s, -2.0)


def _norm_kernel(p_ref, rn_ref):
    rn_ref[...] = lax.rsqrt(jnp.sum(p_ref[...], axis=0, keepdims=True))


def _mlp(features, W1, b1, W2, b2):
    return pl.pallas_call(
        _mlp_kernel,
        grid=(N // _ROW_BLK,),
        in_specs=[
            pl.BlockSpec((_ROW_BLK, NFEAT), lambda i: (i, 0)),
            pl.BlockSpec((NFEAT, NHID), lambda i: (0, 0)),
            pl.BlockSpec((NHID,), lambda i: (0,)),
            pl.BlockSpec((NHID, NOUT), lambda i: (0, 0)),
            pl.BlockSpec((NOUT,), lambda i: (0,)),
        ],
        out_specs=[
            pl.BlockSpec((_ROW_BLK, NOUT), lambda i: (i, 0)),
            pl.BlockSpec((_ROW_BLK, NOUT), lambda i: (i, 0)),
        ],
        out_shape=[
            jax.ShapeDtypeStruct((N, NOUT), jnp.float32),
            jax.ShapeDtypeStruct((N, NOUT), jnp.float32),
        ],
    )(features, W1, b1, W2, b2)


def _sim(xn_pad):
    return pl.pallas_call(
        _sim_kernel,
        grid=(NC // _SIM_BLK,),
        in_specs=[
            pl.BlockSpec((_SIM_BLK, NOUT), lambda i: (i, 0)),
            pl.BlockSpec((NC, NOUT), lambda i: (0, 0)),
        ],
        out_specs=pl.BlockSpec((_SIM_BLK, NC), lambda i: (i, 0)),
        out_shape=jax.ShapeDtypeStruct((NC, NC), jnp.float32),
    )(xn_pad, xn_pad)


def _norm(partials):
    return pl.pallas_call(
        _norm_kernel,
        out_shape=jax.ShapeDtypeStruct((1, NP), jnp.float32),
    )(partials)


def _iota16():
    return lax.iota(jnp.int32, 16)


def _splat(x, dtype):
    return jnp.full((16,), x, dtype=dtype)


def _lane_of(mask):
    return jnp.max(plsc.all_reduce_ffs(mask))


def _extract(vec, l, lowest):
    return jnp.max(jnp.where(_iota16() == l, vec, lowest))


def _pick(row_v, l1t_v, vals_v, inds_v, t, colmax, colarg):
    iota = _iota16()
    lane0 = iota == 0
    m = jnp.max(colmax)
    l = _lane_of(colmax == m)
    g = _extract(colarg, l, 0)
    c = g * 16 + l
    v = row_v[pl.ds(c * 16, 16)]
    k = _lane_of(v == m)
    src = c * 16 + k
    plsc.store_scatter(vals_v, [_splat(t, jnp.int32)],
                       _splat(m, jnp.float32), mask=lane0)
    plsc.store_scatter(inds_v, [_splat(t, jnp.int32)],
                       _splat(src, jnp.int32), mask=lane0)
    vnew = jnp.where(iota == k, -3.0, v)
    row_v[pl.ds(c * 16, 16)] = vnew
    plsc.store_scatter(l1t_v, [_splat(l * _LS + g, jnp.int32)],
                       _splat(jnp.max(vnew), jnp.float32), mask=lane0)
    c0 = l1t_v[pl.ds(l * _LS, 16)]
    c1 = l1t_v[pl.ds(l * _LS + 16, 16)]
    c2 = l1t_v[pl.ds(l * _LS + 32, 16)]
    best = jnp.max(jnp.maximum(jnp.maximum(c0, c1), c2))
    gb = jnp.min(jnp.where(c0 == best, iota,
                 jnp.where(c1 == best, iota + 16,
                 jnp.where(c2 == best, iota + 32, 99))))
    at_l = iota == l
    return jnp.where(at_l, best, colmax), jnp.where(at_l, gb, colarg)


def _accum_degrees(vals_v, inds_v, part_v, r, live):
    iota = _iota16()
    rowsum = jnp.zeros((), jnp.float32)
    for c in range(KP // 16):
        vv = vals_v[pl.ds(c * 16, 16)]
        ii = inds_v[pl.ds(c * 16, 16)]
        valid = jnp.logical_and((c * 16 + iota) < (K + 1), live)
        plsc.addupdate_scatter(part_v, [ii], vv, mask=valid)
        rowsum = rowsum + jnp.sum(jnp.where(valid, vv, 0.0))
    plsc.addupdate_scatter(part_v, [_splat(r, jnp.int32)],
                           _splat(rowsum, jnp.float32),
                           mask=jnp.logical_and(iota == 0, live))


_RIF = 4


def _sc_topk_body(sim_hbm, vals_hbm, inds_hbm, part_hbm, *scr):
    rows = scr[0:_RIF]
    l1ts = scr[_RIF:2 * _RIF]
    valss = scr[2 * _RIF:3 * _RIF]
    indss = scr[3 * _RIF:4 * _RIF]
    part_v = scr[4 * _RIF]
    dma_sem = scr[4 * _RIF + 1]

    wid = lax.axis_index("s") * 2 + lax.axis_index("c")
    zf = jnp.zeros((16,), jnp.float32)
    zi = jnp.zeros((16,), jnp.int32)

    for s in range(_RIF):
        for c in range(KP // 16):
            valss[s][pl.ds(c * 16, 16)] = zf
            indss[s][pl.ds(c * 16, 16)] = zi
        for c in range(0, _L1PAD, 16):
            l1ts[s][pl.ds(c, 16)] = jnp.full((16,), -4.0, jnp.float32)

    def _zero(i, _):
        part_v[pl.ds(i * 16, 16)] = zf
        return 0
    lax.fori_loop(0, NP // 16, _zero, 0)

    def _do_batch(i, wid):
        r = [wid + _NW * (_RIF * i + s) for s in range(_RIF)]
        copies = [pltpu.async_copy(sim_hbm.at[r[s]], rows[s], dma_sem)
                  for s in range(_RIF)]
        for cp in copies:
            cp.wait()

        iota = _iota16()

        def _build(g, carry):
            out = []
            for s in range(_RIF):
                cmax, carg = carry[2 * s], carry[2 * s + 1]
                acc = jnp.full((16,), -4.0, jnp.float32)
                for l in range(16):
                    v = rows[s][pl.ds(g * 256 + l * 16, 16)]
                    acc = jnp.where(iota == l, jnp.max(v), acc)
                plsc.store_scatter(l1ts[s], [iota * _LS + _splat(g, jnp.int32)],
                                   acc)
                upd = acc > cmax
                out.extend((jnp.maximum(cmax, acc), jnp.where(upd, g, carg)))
            return tuple(out)
        st = lax.fori_loop(
            0, _G, _build,
            tuple([jnp.full((16,), -5.0, jnp.float32),
                   jnp.zeros((16,), jnp.int32)] * _RIF))

        def _select(t, carry):
            out = []
            for s in range(_RIF):
                cm, ca = _pick(rows[s], l1ts[s], valss[s], indss[s], t,
                               carry[2 * s], carry[2 * s + 1])
                out.extend((cm, ca))
            return tuple(out)
        lax.fori_loop(0, K + 1, _select, tuple(st))

        for s in range(_RIF):
            live = r[s] < N if s > 0 else True
            _accum_degrees(valss[s], indss[s], part_v, r[s], live)
            if s == 0:
                pltpu.sync_copy(valss[0], vals_hbm.at[r[0]])
                pltpu.sync_copy(indss[0], inds_hbm.at[r[0]])
            else:
                @pl.when(live)
                def _(s=s):
                    pltpu.sync_copy(valss[s], vals_hbm.at[r[s]])
                    pltpu.sync_copy(indss[s], inds_hbm.at[r[s]])
        return wid

    batches = jnp.where(wid < (N - (N // _NW) * _NW),
                        (N // _NW + 1 + _RIF - 1) // _RIF,
                        N // _NW // _RIF)
    lax.fori_loop(0, batches, _do_batch, wid)
    pltpu.sync_copy(part_v, part_hbm.at[wid])


def _sc_topk(sim):
    mesh = plsc.VectorSubcoreMesh(core_axis_name="c", subcore_axis_name="s")
    return pl.kernel(
        _sc_topk_body,
        out_type=[
            jax.ShapeDtypeStruct((N, KP), jnp.float32),
            jax.ShapeDtypeStruct((N, KP), jnp.int32),
            jax.ShapeDtypeStruct((_NW, NP), jnp.float32),
        ],
        mesh=mesh,
        compiler_params=pltpu.CompilerParams(needs_layout_passes=False),
        scratch_types=(
            [pltpu.VMEM((NC,), jnp.float32)] * _RIF
            + [pltpu.VMEM((_L1PAD,), jnp.float32)] * _RIF
            + [pltpu.VMEM((KP,), jnp.float32)] * _RIF
            + [pltpu.VMEM((KP,), jnp.int32)] * _RIF
            + [pltpu.VMEM((NP,), jnp.float32)]
            + [pltpu.SemaphoreType.DMA]
        ),
    )(sim)


_EB = 8
_EBAND = 312


def _sc_edgeval_body(vals_hbm, inds_hbm, rn_hbm, out_hbm,
                     rn_v, vv_v, iv_v, ov_v, vv1_v, iv1_v, ov1_v):
    wid = lax.axis_index("s") * 2 + lax.axis_index("c")
    pltpu.sync_copy(rn_hbm.at[0], rn_v)
    lo = wid * _EBAND

    def _scale_row(vv, iv, ov, i, r):
        rr = plsc.load_gather(rn_v, [_splat(r, jnp.int32)])
        for c in range(KP // 16):
            if i is None:
                v = vv[pl.ds(c * 16, 16)]
                ii = iv[pl.ds(c * 16, 16)]
            else:
                v = vv[i, pl.ds(c * 16, 16)]
                ii = iv[i, pl.ds(c * 16, 16)]
            rc = plsc.load_gather(rn_v, [ii])
            if i is None:
                ov[pl.ds(c * 16, 16)] = v * rr * rc
            else:
                ov[i, pl.ds(c * 16, 16)] = v * rr * rc

    def _do_block(b, lo):
        base = lo + _EB * b
        pltpu.sync_copy(vals_hbm.at[pl.ds(base, _EB)], vv_v)
        pltpu.sync_copy(inds_hbm.at[pl.ds(base, _EB)], iv_v)
        for i in range(_EB):
            _scale_row(vv_v, iv_v, ov_v, i, base + i)
        pltpu.sync_copy(ov_v, out_hbm.at[pl.ds(base, _EB)])
        return lo

    lax.fori_loop(0, _EBAND // _EB, _do_block, lo)

    @pl.when(wid < N - _NW * _EBAND)
    def _():
        r = _NW * _EBAND + wid
        pltpu.sync_copy(vals_hbm.at[r], vv1_v)
        pltpu.sync_copy(inds_hbm.at[r], iv1_v)
        _scale_row(vv1_v, iv1_v, ov1_v, None, r)
        pltpu.sync_copy(ov1_v, out_hbm.at[r])


def _sc_edgeval(vals, inds, rnorm):
    mesh = plsc.VectorSubcoreMesh(core_axis_name="c", subcore_axis_name="s")
    return pl.kernel(
        _sc_edgeval_body,
        out_type=jax.ShapeDtypeStruct((N, KP), jnp.float32),
        mesh=mesh,
        compiler_params=pltpu.CompilerParams(needs_layout_passes=False),
        scratch_types=[
            pltpu.VMEM((NP,), jnp.float32),
            pltpu.VMEM((_EB, KP), jnp.float32),
            pltpu.VMEM((_EB, KP), jnp.int32),
            pltpu.VMEM((_EB, KP), jnp.float32),
            pltpu.VMEM((KP,), jnp.float32),
            pltpu.VMEM((KP,), jnp.int32),
            pltpu.VMEM((KP,), jnp.float32),
        ],
    )(vals, inds, rnorm)


def kernel(features, W1, b1, W2, b2):
    emb, xn = _mlp(features, W1, b1, W2, b2)
    xn_pad = jnp.pad(xn, ((0, NC - N), (0, 0)))
    sim = _sim(xn_pad)
    vals, inds, partials = _sc_topk(sim)
    rnorm = _norm(partials)
    values = _sc_edgeval(vals, inds, rnorm)

    rows = jnp.repeat(jnp.arange(N, dtype=jnp.int32), K + 1)
    cols = inds[:, :K + 1].reshape(-1)
    values = values[:, :K + 1].reshape(-1)
    edge_index = jnp.stack([rows.astype(jnp.int64), cols.astype(jnp.int64)],
                           axis=0)
    return edge_index, values, emb


# speedup vs baseline: 1.2374x; 1.1069x over previous
import functools

import jax
import jax.numpy as jnp
from jax import lax
from jax.experimental import pallas as pl
from jax.experimental.pallas import tpu as pltpu
from jax.experimental.pallas import tpu_sc as plsc

N = 10000
NFEAT = 512
NHID = 128
NOUT = 64
K = 100
KP = 112
NP = 10016
NC = 10240

_ROW_BLK = 2000
_SIM_BLK = 512

_NW = 32
_G = 40
_LS = 49
_L1PAD = 800


def _mlp_kernel(f_ref, w1_ref, b1_ref, w2_ref, b2_ref, emb_ref, xn_ref):
    h = jnp.maximum(
        jnp.dot(f_ref[...], w1_ref[...], preferred_element_type=jnp.float32)
        + b1_ref[...], 0.0)
    emb = jnp.dot(h, w2_ref[...], preferred_element_type=jnp.float32) + b2_ref[...]
    emb_ref[...] = emb
    nrm = jnp.sqrt(jnp.sum(emb * emb, axis=1, keepdims=True))
    xn_ref[...] = emb / jnp.clip(nrm, 1e-12, None)


def _sim_kernel(xb_ref, xf_ref, sim_ref):
    s = lax.dot_general(
        xb_ref[...], xf_ref[...],
        dimension_numbers=(((1,), (1,)), ((), ())),
        preferred_element_type=jnp.float32)
    col = lax.broadcasted_iota(jnp.int32, s.shape, 1)
    sim_ref[...] = jnp.where(col < N, s, -2.0)


def _norm_kernel(p_ref, rn_ref):
    rn_ref[...] = lax.rsqrt(jnp.sum(p_ref[...], axis=0, keepdims=True))


def _mlp(features, W1, b1, W2, b2):
    return pl.pallas_call(
        _mlp_kernel,
        grid=(N // _ROW_BLK,),
        in_specs=[
            pl.BlockSpec((_ROW_BLK, NFEAT), lambda i: (i, 0)),
            pl.BlockSpec((NFEAT, NHID), lambda i: (0, 0)),
            pl.BlockSpec((NHID,), lambda i: (0,)),
            pl.BlockSpec((NHID, NOUT), lambda i: (0, 0)),
            pl.BlockSpec((NOUT,), lambda i: (0,)),
        ],
        out_specs=[
            pl.BlockSpec((_ROW_BLK, NOUT), lambda i: (i, 0)),
            pl.BlockSpec((_ROW_BLK, NOUT), lambda i: (i, 0)),
        ],
        out_shape=[
            jax.ShapeDtypeStruct((N, NOUT), jnp.float32),
            jax.ShapeDtypeStruct((N, NOUT), jnp.float32),
        ],
    )(features, W1, b1, W2, b2)


def _sim(xn_pad):
    return pl.pallas_call(
        _sim_kernel,
        grid=(NC // _SIM_BLK,),
        in_specs=[
            pl.BlockSpec((_SIM_BLK, NOUT), lambda i: (i, 0)),
            pl.BlockSpec((NC, NOUT), lambda i: (0, 0)),
        ],
        out_specs=pl.BlockSpec((_SIM_BLK, NC), lambda i: (i, 0)),
        out_shape=jax.ShapeDtypeStruct((NC, NC), jnp.float32),
    )(xn_pad, xn_pad)


def _norm(partials):
    return pl.pallas_call(
        _norm_kernel,
        out_shape=jax.ShapeDtypeStruct((1, NP), jnp.float32),
    )(partials)


def _iota16():
    return lax.iota(jnp.int32, 16)


def _splat(x, dtype):
    return jnp.full((16,), x, dtype=dtype)


def _lane_of(mask):
    return jnp.max(plsc.all_reduce_ffs(mask))


def _extract(vec, l, lowest):
    return jnp.max(jnp.where(_iota16() == l, vec, lowest))


def _pick(row_v, l1t_v, vals_v, inds_v, t, colmax, colarg):
    iota = _iota16()
    lane0 = iota == 0
    m = jnp.max(colmax)
    l = _lane_of(colmax == m)
    g = _extract(colarg, l, 0)
    c = g * 16 + l
    v = row_v[pl.ds(c * 16, 16)]
    k = _lane_of(v == m)
    src = c * 16 + k
    plsc.store_scatter(vals_v, [_splat(t, jnp.int32)],
                       _splat(m, jnp.float32), mask=lane0)
    plsc.store_scatter(inds_v, [_splat(t, jnp.int32)],
                       _splat(src, jnp.int32), mask=lane0)
    vnew = jnp.where(iota == k, -3.0, v)
    row_v[pl.ds(c * 16, 16)] = vnew
    plsc.store_scatter(l1t_v, [_splat(l * _LS + g, jnp.int32)],
                       _splat(jnp.max(vnew), jnp.float32), mask=lane0)
    c0 = l1t_v[pl.ds(l * _LS, 16)]
    c1 = l1t_v[pl.ds(l * _LS + 16, 16)]
    c2 = l1t_v[pl.ds(l * _LS + 32, 16)]
    best = jnp.max(jnp.maximum(jnp.maximum(c0, c1), c2))
    gb = jnp.min(jnp.where(c0 == best, iota,
                 jnp.where(c1 == best, iota + 16,
                 jnp.where(c2 == best, iota + 32, 99))))
    at_l = iota == l
    return jnp.where(at_l, best, colmax), jnp.where(at_l, gb, colarg)


def _accum_degrees(vals_v, inds_v, part_v, r, live):
    iota = _iota16()
    rowsum = jnp.zeros((), jnp.float32)
    for c in range(KP // 16):
        vv = vals_v[pl.ds(c * 16, 16)]
        ii = inds_v[pl.ds(c * 16, 16)]
        valid = jnp.logical_and((c * 16 + iota) < (K + 1), live)
        plsc.addupdate_scatter(part_v, [ii], vv, mask=valid)
        rowsum = rowsum + jnp.sum(jnp.where(valid, vv, 0.0))
    plsc.addupdate_scatter(part_v, [_splat(r, jnp.int32)],
                           _splat(rowsum, jnp.float32),
                           mask=jnp.logical_and(iota == 0, live))


_RIF = 4


def _sc_topk_body(sim_hbm, vals_hbm, inds_hbm, part_hbm, *scr):
    rows_ab = (scr[0:_RIF], scr[_RIF:2 * _RIF])
    l1ts = scr[2 * _RIF:3 * _RIF]
    valss = scr[3 * _RIF:4 * _RIF]
    indss = scr[4 * _RIF:5 * _RIF]
    part_v = scr[5 * _RIF]
    sem_ab = (scr[5 * _RIF + 1], scr[5 * _RIF + 2])

    wid = lax.axis_index("s") * 2 + lax.axis_index("c")
    zf = jnp.zeros((16,), jnp.float32)
    zi = jnp.zeros((16,), jnp.int32)

    def _rows_of(i):
        return [wid + _NW * (_RIF * i + s) for s in range(_RIF)]

    def _issue(bufs, i, sem):
        for s, rr in enumerate(_rows_of(i)):
            pltpu.async_copy(sim_hbm.at[rr], bufs[s], sem)

    def _wait(bufs, i, sem):
        for s, rr in enumerate(_rows_of(i)):
            pltpu.make_async_copy(sim_hbm.at[rr], bufs[s], sem).wait()

    _issue(rows_ab[0], 0, sem_ab[0])

    for s in range(_RIF):
        for c in range(KP // 16):
            valss[s][pl.ds(c * 16, 16)] = zf
            indss[s][pl.ds(c * 16, 16)] = zi
        for c in range(0, _L1PAD, 16):
            l1ts[s][pl.ds(c, 16)] = jnp.full((16,), -4.0, jnp.float32)

    def _zero(i, _):
        part_v[pl.ds(i * 16, 16)] = zf
        return 0
    lax.fori_loop(0, NP // 16, _zero, 0)

    batches = jnp.where(wid < (N - (N // _NW) * _NW),
                        (N // _NW + 1 + _RIF - 1) // _RIF,
                        N // _NW // _RIF)

    def _process(rows, i):
        r = _rows_of(i)
        iota = _iota16()

        def _build(g, carry):
            out = []
            for s in range(_RIF):
                cmax, carg = carry[2 * s], carry[2 * s + 1]
                acc = jnp.full((16,), -4.0, jnp.float32)
                for l in range(16):
                    v = rows[s][pl.ds(g * 256 + l * 16, 16)]
                    acc = jnp.where(iota == l, jnp.max(v), acc)
                plsc.store_scatter(l1ts[s], [iota * _LS + _splat(g, jnp.int32)],
                                   acc)
                upd = acc > cmax
                out.extend((jnp.maximum(cmax, acc), jnp.where(upd, g, carg)))
            return tuple(out)
        st = lax.fori_loop(
            0, _G, _build,
            tuple([jnp.full((16,), -5.0, jnp.float32),
                   jnp.zeros((16,), jnp.int32)] * _RIF))

        def _select(t, carry):
            out = []
            for s in range(_RIF):
                cm, ca = _pick(rows[s], l1ts[s], valss[s], indss[s], t,
                               carry[2 * s], carry[2 * s + 1])
                out.extend((cm, ca))
            return tuple(out)
        lax.fori_loop(0, K + 1, _select, tuple(st))

        for s in range(_RIF):
            live = r[s] < N
            _accum_degrees(valss[s], indss[s], part_v, r[s], live)

            @pl.when(live)
            def _(s=s):
                pltpu.sync_copy(valss[s], vals_hbm.at[r[s]])
                pltpu.sync_copy(indss[s], inds_hbm.at[r[s]])

    def _do_pair(p, wid):
        i_a = 2 * p
        i_b = i_a + 1
        _issue(rows_ab[1], i_b, sem_ab[1])
        _wait(rows_ab[0], i_a, sem_ab[0])
        _process(rows_ab[0], i_a)

        @pl.when(i_a + 2 < batches)
        def _():
            _issue(rows_ab[0], i_a + 2, sem_ab[0])
        _wait(rows_ab[1], i_b, sem_ab[1])
        _process(rows_ab[1], i_b)
        return wid

    lax.fori_loop(0, (batches + 1) // 2, _do_pair, wid)
    pltpu.sync_copy(part_v, part_hbm.at[wid])


def _sc_topk(sim):
    mesh = plsc.VectorSubcoreMesh(core_axis_name="c", subcore_axis_name="s")
    return pl.kernel(
        _sc_topk_body,
        out_type=[
            jax.ShapeDtypeStruct((N, KP), jnp.float32),
            jax.ShapeDtypeStruct((N, KP), jnp.int32),
            jax.ShapeDtypeStruct((_NW, NP), jnp.float32),
        ],
        mesh=mesh,
        compiler_params=pltpu.CompilerParams(needs_layout_passes=False),
        scratch_types=(
            [pltpu.VMEM((NC,), jnp.float32)] * (2 * _RIF)
            + [pltpu.VMEM((_L1PAD,), jnp.float32)] * _RIF
            + [pltpu.VMEM((KP,), jnp.float32)] * _RIF
            + [pltpu.VMEM((KP,), jnp.int32)] * _RIF
            + [pltpu.VMEM((NP,), jnp.float32)]
            + [pltpu.SemaphoreType.DMA, pltpu.SemaphoreType.DMA]
        ),
    )(sim)


_EB = 8
_EBAND = 312


def _sc_edgeval_body(vals_hbm, inds_hbm, rn_hbm, out_hbm,
                     rn_v, vv_v, iv_v, ov_v, vv1_v, iv1_v, ov1_v):
    wid = lax.axis_index("s") * 2 + lax.axis_index("c")
    pltpu.sync_copy(rn_hbm.at[0], rn_v)
    lo = wid * _EBAND

    def _scale_row(vv, iv, ov, i, r):
        rr = plsc.load_gather(rn_v, [_splat(r, jnp.int32)])
        for c in range(KP // 16):
            if i is None:
                v = vv[pl.ds(c * 16, 16)]
                ii = iv[pl.ds(c * 16, 16)]
            else:
                v = vv[i, pl.ds(c * 16, 16)]
                ii = iv[i, pl.ds(c * 16, 16)]
            rc = plsc.load_gather(rn_v, [ii])
            if i is None:
                ov[pl.ds(c * 16, 16)] = v * rr * rc
            else:
                ov[i, pl.ds(c * 16, 16)] = v * rr * rc

    def _do_block(b, lo):
        base = lo + _EB * b
        pltpu.sync_copy(vals_hbm.at[pl.ds(base, _EB)], vv_v)
        pltpu.sync_copy(inds_hbm.at[pl.ds(base, _EB)], iv_v)
        for i in range(_EB):
            _scale_row(vv_v, iv_v, ov_v, i, base + i)
        pltpu.sync_copy(ov_v, out_hbm.at[pl.ds(base, _EB)])
        return lo

    lax.fori_loop(0, _EBAND // _EB, _do_block, lo)

    @pl.when(wid < N - _NW * _EBAND)
    def _():
        r = _NW * _EBAND + wid
        pltpu.sync_copy(vals_hbm.at[r], vv1_v)
        pltpu.sync_copy(inds_hbm.at[r], iv1_v)
        _scale_row(vv1_v, iv1_v, ov1_v, None, r)
        pltpu.sync_copy(ov1_v, out_hbm.at[r])


def _sc_edgeval(vals, inds, rnorm):
    mesh = plsc.VectorSubcoreMesh(core_axis_name="c", subcore_axis_name="s")
    return pl.kernel(
        _sc_edgeval_body,
        out_type=jax.ShapeDtypeStruct((N, KP), jnp.float32),
        mesh=mesh,
        compiler_params=pltpu.CompilerParams(needs_layout_passes=False),
        scratch_types=[
            pltpu.VMEM((NP,), jnp.float32),
            pltpu.VMEM((_EB, KP), jnp.float32),
            pltpu.VMEM((_EB, KP), jnp.int32),
            pltpu.VMEM((_EB, KP), jnp.float32),
            pltpu.VMEM((KP,), jnp.float32),
            pltpu.VMEM((KP,), jnp.int32),
            pltpu.VMEM((KP,), jnp.float32),
        ],
    )(vals, inds, rnorm)


def kernel(features, W1, b1, W2, b2):
    emb, xn = _mlp(features, W1, b1, W2, b2)
    xn_pad = jnp.pad(xn, ((0, NC - N), (0, 0)))
    sim = _sim(xn_pad)
    vals, inds, partials = _sc_topk(sim)
    rnorm = _norm(partials)
    values = _sc_edgeval(vals, inds, rnorm)

    rows = jnp.repeat(jnp.arange(N, dtype=jnp.int32), K + 1)
    cols = inds[:, :K + 1].reshape(-1)
    values = values[:, :K + 1].reshape(-1)
    edge_index = jnp.stack([rows.astype(jnp.int64), cols.astype(jnp.int64)],
                           axis=0)
    return edge_index, values, emb
